# baseline (speedup 1.0000x reference)
import jax
import jax.numpy as jnp
from jax import lax
from jax.experimental import pallas as pl
from jax.experimental.pallas import tpu as pltpu
from jax.experimental.pallas import tpu_sc as plsc

_B, _H, _W = 4, 512, 512
_GD, _GH, _GW = 8, 16, 16
_C12 = 12

_H_SC = 448
_H_TC = _H - _H_SC

_ROWS = 16


def _strip_j0(s):
    return jnp.clip((s - 1) // 2, 0, _GH - 2)


def _tc_kernel(g0_ref, g1_ref, guide_ref, img_ref, out_ref):
    s = pl.program_id(1)
    f32 = jnp.float32

    i_idx = jax.lax.broadcasted_iota(jnp.int32, (_GW, _W), 0).astype(f32)
    x_idx = jax.lax.broadcasted_iota(jnp.int32, (_GW, _W), 1).astype(f32)
    gx = jnp.clip((x_idx + 0.5) * (_GW / _W), 0.5, _GW - 0.5)
    xwt = jnp.maximum(1.0 - jnp.abs(i_idx + 0.5 - gx), 0.0)
    dn = (((1,), (0,)), ((), ()))
    u0 = jax.lax.dot_general(g0_ref[0, 0], xwt, dn,
                             precision=jax.lax.Precision.HIGHEST,
                             preferred_element_type=f32)
    u1 = jax.lax.dot_general(g1_ref[0, 0], xwt, dn,
                             precision=jax.lax.Precision.HIGHEST,
                             preferred_element_type=f32)

    r = jax.lax.broadcasted_iota(jnp.int32, (_ROWS, 1), 0).astype(f32)
    y = s.astype(f32) * _ROWS + r
    gy = jnp.clip((y + 0.5) * (_GH / _H), 0.5, _GH - 0.5)
    fy = jnp.clip(jnp.floor(gy - 0.5), 0.0, _GH - 2.0)
    wy1 = gy - 0.5 - fy
    wy0 = 1.0 - wy1

    gz = jnp.clip(guide_ref[0] * _GD, 0.5, _GD - 0.5)

    acc = [None] * _C12
    for z in range(_GD):
        zw = jnp.maximum(1.0 - jnp.abs(z + 0.5 - gz), 0.0)
        a0 = zw * wy0
        a1 = zw * wy1
        for c in range(_C12):
            row = z * _C12 + c
            t = a0 * u0[row] + a1 * u1[row]
            acc[c] = t if acc[c] is None else acc[c] + t

    img = img_ref[0]
    for o in range(3):
        res = (acc[4 * o + 0] * img[0] + acc[4 * o + 1] * img[1]
               + acc[4 * o + 2] * img[2] + acc[4 * o + 3])
        out_ref[0, o] = res


def _tc_part(g3, guide, image):
    return pl.pallas_call(
        _tc_kernel,
        grid=(_B, _H_TC // _ROWS),
        in_specs=[
            pl.BlockSpec((1, 1, _GD * _C12, _GW),
                         lambda b, s: (b, _strip_j0(s), 0, 0)),
            pl.BlockSpec((1, 1, _GD * _C12, _GW),
                         lambda b, s: (b, _strip_j0(s) + 1, 0, 0)),
            pl.BlockSpec((1, _ROWS, _W), lambda b, s: (b, s, 0)),
            pl.BlockSpec((1, 3, _ROWS, _W), lambda b, s: (b, 0, s, 0)),
        ],
        out_specs=pl.BlockSpec((1, 3, _ROWS, _W), lambda b, s: (b, 0, s, 0)),
        out_shape=jax.ShapeDtypeStruct((_B, 3, _H, _W), jnp.float32),
        compiler_params=pltpu.CompilerParams(
            dimension_semantics=("parallel", "parallel")),
    )(g3, g3, guide, image)


_ZIC = _GD * _GW * _C12
_NW = 32
_RPW = (_B * _H_SC) // _NW
_CHUNK = 8
_CW = _CHUNK * _W
_NV = _W // 16


def _sc_body(grid_hbm, guide_hbm, image_hbm, out_hbm,
             g_v, gy_v, gd_v, im_v, ob_v):
    f32, i32 = jnp.float32, jnp.int32
    wid = lax.axis_index("s") * 2 + lax.axis_index("c")
    row0 = wid * _RPW
    b = row0 // _H_SC

    pltpu.sync_copy(grid_hbm.at[pl.ds(b * _GH * _ZIC, _GH * _ZIC)], g_v)

    def chunk_body(k, _):
        ysc = row0 % _H_SC + k * _CHUNK
        y0 = _H_TC + ysc
        base_px = (b * _H + y0) * _W
        pltpu.sync_copy(guide_hbm.at[pl.ds(base_px, _CW)], gd_v)
        for ci in range(3):
            pltpu.sync_copy(
                image_hbm.at[pl.ds(((b * 3 + ci) * _H + y0) * _W, _CW)],
                im_v.at[pl.ds(ci * _CW, _CW)])

        def row_body(r, _):
            y = y0 + r
            j0 = jnp.clip((y // 16 - 1) // 2, 0, _GH - 2)
            gy = jnp.clip((y.astype(f32) + 0.5) * (_GH / _H), 0.5, _GH - 0.5)
            wy1 = gy - 0.5 - j0.astype(f32)
            wy1v = jnp.full((16,), wy1, f32)
            wy0v = 1.0 - wy1v
            goff = j0 * _ZIC

            def prep_body(k2, _):
                o = k2 * 16
                gy_v[pl.ds(o, 16)] = (
                    wy0v * g_v[pl.ds(goff + o, 16)]
                    + wy1v * g_v[pl.ds(goff + _ZIC + o, 16)])
                return ()
            lax.fori_loop(0, _ZIC // 16, prep_body, (), unroll=8)

            lane = lax.iota(i32, 16)
            roff = r * _W

            def vec_body(v, _):
                px = pl.ds(roff + v * 16, 16)
                i0 = jnp.clip((v - 1) // 2, 0, _GW - 2)
                x = (v * 16 + lane).astype(f32)
                gx = jnp.clip((x + 0.5) * (_GW / _W), 0.5, _GW - 0.5)
                wx1 = gx - 0.5 - i0.astype(f32)
                wx0 = 1.0 - wx1

                g = gd_v[px]
                gz = jnp.clip(g * _GD, 0.5, _GD - 0.5)
                u = gz - 0.5
                fz = jnp.minimum(u.astype(i32), _GD - 2)
                t = u - fz.astype(f32)
                s = 1.0 - t

                base = fz + i0 * (_C12 * _GD)
                co = []
                for c in range(12):
                    g00 = plsc.load_gather(gy_v, [base + c * _GD])
                    g01 = plsc.load_gather(gy_v, [base + (c * _GD + 1)])
                    g10 = plsc.load_gather(gy_v, [base + (c * _GD + _C12 * _GD)])
                    g11 = plsc.load_gather(gy_v, [base + (c * _GD + _C12 * _GD + 1)])
                    v0 = s * g00 + t * g01
                    v1 = s * g10 + t * g11
                    co.append(wx0 * v0 + wx1 * v1)

                p0 = im_v[pl.ds(0 * _CW + roff + v * 16, 16)]
                p1 = im_v[pl.ds(1 * _CW + roff + v * 16, 16)]
                p2 = im_v[pl.ds(2 * _CW + roff + v * 16, 16)]
                for o in range(3):
                    res = (co[4 * o] * p0 + co[4 * o + 1] * p1
                           + co[4 * o + 2] * p2 + co[4 * o + 3])
                    ob_v[pl.ds(o * _CW + roff + v * 16, 16)] = res
                return ()
            lax.fori_loop(0, _NV, vec_body, (), unroll=4)
            return ()
        lax.fori_loop(0, _CHUNK, row_body, ())

        for o in range(3):
            pltpu.sync_copy(
                ob_v.at[pl.ds(o * _CW, _CW)],
                out_hbm.at[pl.ds(((b * 3 + o) * _H_SC + ysc) * _W, _CW)])
        return ()
    lax.fori_loop(0, _RPW // _CHUNK, chunk_body, ())


def _sc_part(g_sc, guide_f, image_f):
    mesh = plsc.VectorSubcoreMesh(core_axis_name="c", subcore_axis_name="s")
    run = pl.kernel(
        _sc_body,
        out_type=jax.ShapeDtypeStruct((_B * 3 * _H_SC * _W,), jnp.float32),
        mesh=mesh,
        compiler_params=pltpu.CompilerParams(needs_layout_passes=False),
        scratch_types=[
            pltpu.VMEM((_GH * _ZIC,), jnp.float32),
            pltpu.VMEM((_ZIC,), jnp.float32),
            pltpu.VMEM((_CW,), jnp.float32),
            pltpu.VMEM((3 * _CW,), jnp.float32),
            pltpu.VMEM((3 * _CW,), jnp.float32),
        ],
    )
    return run(g_sc, guide_f, image_f)


@jax.jit
def kernel(grid, guide, image):
    B, C12, gd, gh, gw = grid.shape
    g3 = jnp.transpose(grid, (0, 3, 2, 1, 4)).reshape(B, gh, gd * C12, gw)
    g_sc = jnp.transpose(grid, (0, 3, 4, 1, 2)).reshape(B * gh * gd * gw * C12)
    guide_f = guide.reshape(B * _H * _W)
    image_f = image.reshape(B * 3 * _H * _W)

    tc_out = _tc_part(g3, guide, image)
    sc_out = _sc_part(g_sc, guide_f, image_f).reshape(B, 3, _H_SC, _W)
    return lax.dynamic_update_slice(tc_out, sc_out, (0, 0, _H_TC, 0))


# speedup vs baseline: 1.6678x; 1.6678x over previous
import jax
import jax.numpy as jnp
from jax import lax
from jax.experimental import pallas as pl
from jax.experimental.pallas import tpu as pltpu
from jax.experimental.pallas import tpu_sc as plsc

_B, _H, _W = 4, 512, 512
_GD, _GH, _GW = 8, 16, 16
_C12 = 12

_H_SC = 192
_H_TC = _H - _H_SC

_ROWS = 16


def _strip_j0(s):
    return jnp.clip((s - 1) // 2, 0, _GH - 2)


def _tc_kernel(g0_ref, g1_ref, guide_ref, img_ref, out_ref):
    s = pl.program_id(1)
    f32 = jnp.float32

    i_idx = jax.lax.broadcasted_iota(jnp.int32, (_GW, _W), 0).astype(f32)
    x_idx = jax.lax.broadcasted_iota(jnp.int32, (_GW, _W), 1).astype(f32)
    gx = jnp.clip((x_idx + 0.5) * (_GW / _W), 0.5, _GW - 0.5)
    xwt = jnp.maximum(1.0 - jnp.abs(i_idx + 0.5 - gx), 0.0)
    dn = (((1,), (0,)), ((), ()))
    u0 = jax.lax.dot_general(g0_ref[0, 0], xwt, dn,
                             precision=jax.lax.Precision.HIGHEST,
                             preferred_element_type=f32)
    u1 = jax.lax.dot_general(g1_ref[0, 0], xwt, dn,
                             precision=jax.lax.Precision.HIGHEST,
                             preferred_element_type=f32)

    r = jax.lax.broadcasted_iota(jnp.int32, (_ROWS, 1), 0).astype(f32)
    y = s.astype(f32) * _ROWS + r
    gy = jnp.clip((y + 0.5) * (_GH / _H), 0.5, _GH - 0.5)
    fy = jnp.clip(jnp.floor(gy - 0.5), 0.0, _GH - 2.0)
    wy1 = gy - 0.5 - fy
    wy0 = 1.0 - wy1

    gz = jnp.clip(guide_ref[0] * _GD, 0.5, _GD - 0.5)

    acc = [None] * _C12
    for z in range(_GD):
        zw = jnp.maximum(1.0 - jnp.abs(z + 0.5 - gz), 0.0)
        a0 = zw * wy0
        a1 = zw * wy1
        for c in range(_C12):
            row = z * _C12 + c
            t = a0 * u0[row] + a1 * u1[row]
            acc[c] = t if acc[c] is None else acc[c] + t

    img = img_ref[0]
    for o in range(3):
        res = (acc[4 * o + 0] * img[0] + acc[4 * o + 1] * img[1]
               + acc[4 * o + 2] * img[2] + acc[4 * o + 3])
        out_ref[0, o] = res


def _tc_part(g3, guide, image):
    return pl.pallas_call(
        _tc_kernel,
        grid=(_B, _H_TC // _ROWS),
        in_specs=[
            pl.BlockSpec((1, 1, _GD * _C12, _GW),
                         lambda b, s: (b, _strip_j0(s), 0, 0)),
            pl.BlockSpec((1, 1, _GD * _C12, _GW),
                         lambda b, s: (b, _strip_j0(s) + 1, 0, 0)),
            pl.BlockSpec((1, _ROWS, _W), lambda b, s: (b, s, 0)),
            pl.BlockSpec((1, 3, _ROWS, _W), lambda b, s: (b, 0, s, 0)),
        ],
        out_specs=pl.BlockSpec((1, 3, _ROWS, _W), lambda b, s: (b, 0, s, 0)),
        out_shape=jax.ShapeDtypeStruct((_B, 3, _H, _W), jnp.float32),
        compiler_params=pltpu.CompilerParams(
            dimension_semantics=("parallel", "parallel")),
    )(g3, g3, guide, image)


_ZIC = _GD * _GW * _C12
_NW = 32
_RPW = (_B * _H_SC) // _NW
_CHUNK = 8
_CW = _CHUNK * _W
_NV = _W // 16


def _sc_body(grid_hbm, guide_hbm, image_hbm, out_hbm,
             g_v, gy_v, gd_v, im_v, ob_v):
    f32, i32 = jnp.float32, jnp.int32
    wid = lax.axis_index("s") * 2 + lax.axis_index("c")
    row0 = wid * _RPW
    b = row0 // _H_SC

    pltpu.sync_copy(grid_hbm.at[pl.ds(b * _GH * _ZIC, _GH * _ZIC)], g_v)

    def chunk_body(k, _):
        ysc = row0 % _H_SC + k * _CHUNK
        y0 = _H_TC + ysc
        base_px = (b * _H + y0) * _W
        pltpu.sync_copy(guide_hbm.at[pl.ds(base_px, _CW)], gd_v)
        for ci in range(3):
            pltpu.sync_copy(
                image_hbm.at[pl.ds(((b * 3 + ci) * _H + y0) * _W, _CW)],
                im_v.at[pl.ds(ci * _CW, _CW)])

        def row_body(r, _):
            y = y0 + r
            j0 = jnp.clip((y // 16 - 1) // 2, 0, _GH - 2)
            gy = jnp.clip((y.astype(f32) + 0.5) * (_GH / _H), 0.5, _GH - 0.5)
            wy1 = gy - 0.5 - j0.astype(f32)
            wy1v = jnp.full((16,), wy1, f32)
            wy0v = 1.0 - wy1v
            goff = j0 * _ZIC

            def prep_body(k2, _):
                o = k2 * 16
                gy_v[pl.ds(o, 16)] = (
                    wy0v * g_v[pl.ds(goff + o, 16)]
                    + wy1v * g_v[pl.ds(goff + _ZIC + o, 16)])
                return ()
            lax.fori_loop(0, _ZIC // 16, prep_body, (), unroll=8)

            lane = lax.iota(i32, 16)
            roff = r * _W

            def vec_body(v, _):
                px = pl.ds(roff + v * 16, 16)
                i0 = jnp.clip((v - 1) // 2, 0, _GW - 2)
                x = (v * 16 + lane).astype(f32)
                gx = jnp.clip((x + 0.5) * (_GW / _W), 0.5, _GW - 0.5)
                wx1 = gx - 0.5 - i0.astype(f32)
                wx0 = 1.0 - wx1

                g = gd_v[px]
                gz = jnp.clip(g * _GD, 0.5, _GD - 0.5)
                u = gz - 0.5
                fz = jnp.minimum(u.astype(i32), _GD - 2)
                t = u - fz.astype(f32)
                s = 1.0 - t

                base = fz + i0 * (_C12 * _GD)
                co = []
                for c in range(12):
                    g00 = plsc.load_gather(gy_v, [base + c * _GD])
                    g01 = plsc.load_gather(gy_v, [base + (c * _GD + 1)])
                    g10 = plsc.load_gather(gy_v, [base + (c * _GD + _C12 * _GD)])
                    g11 = plsc.load_gather(gy_v, [base + (c * _GD + _C12 * _GD + 1)])
                    v0 = s * g00 + t * g01
                    v1 = s * g10 + t * g11
                    co.append(wx0 * v0 + wx1 * v1)

                p0 = im_v[pl.ds(0 * _CW + roff + v * 16, 16)]
                p1 = im_v[pl.ds(1 * _CW + roff + v * 16, 16)]
                p2 = im_v[pl.ds(2 * _CW + roff + v * 16, 16)]
                for o in range(3):
                    res = (co[4 * o] * p0 + co[4 * o + 1] * p1
                           + co[4 * o + 2] * p2 + co[4 * o + 3])
                    ob_v[pl.ds(o * _CW + roff + v * 16, 16)] = res
                return ()
            lax.fori_loop(0, _NV, vec_body, (), unroll=4)
            return ()
        lax.fori_loop(0, _CHUNK, row_body, ())

        for o in range(3):
            pltpu.sync_copy(
                ob_v.at[pl.ds(o * _CW, _CW)],
                out_hbm.at[pl.ds(((b * 3 + o) * _H_SC + ysc) * _W, _CW)])
        return ()
    lax.fori_loop(0, _RPW // _CHUNK, chunk_body, ())


def _sc_part(g_sc, guide_f, image_f):
    mesh = plsc.VectorSubcoreMesh(core_axis_name="c", subcore_axis_name="s")
    run = pl.kernel(
        _sc_body,
        out_type=jax.ShapeDtypeStruct((_B * 3 * _H_SC * _W,), jnp.float32),
        mesh=mesh,
        compiler_params=pltpu.CompilerParams(needs_layout_passes=False),
        scratch_types=[
            pltpu.VMEM((_GH * _ZIC,), jnp.float32),
            pltpu.VMEM((_ZIC,), jnp.float32),
            pltpu.VMEM((_CW,), jnp.float32),
            pltpu.VMEM((3 * _CW,), jnp.float32),
            pltpu.VMEM((3 * _CW,), jnp.float32),
        ],
    )
    return run(g_sc, guide_f, image_f)


@jax.jit
def kernel(grid, guide, image):
    B, C12, gd, gh, gw = grid.shape
    g3 = jnp.transpose(grid, (0, 3, 2, 1, 4)).reshape(B, gh, gd * C12, gw)
    g_sc = jnp.transpose(grid, (0, 3, 4, 1, 2)).reshape(B * gh * gd * gw * C12)
    guide_f = guide.reshape(B * _H * _W)
    image_f = image.reshape(B * 3 * _H * _W)

    tc_out = _tc_part(g3, guide, image)
    sc_out = _sc_part(g_sc, guide_f, image_f).reshape(B, 3, _H_SC, _W)
    return lax.dynamic_update_slice(tc_out, sc_out, (0, 0, _H_TC, 0))


# speedup vs baseline: 1.6977x; 1.0179x over previous
import jax
import jax.numpy as jnp
from jax import lax
from jax.experimental import pallas as pl
from jax.experimental.pallas import tpu as pltpu
from jax.experimental.pallas import tpu_sc as plsc

_B, _H, _W = 4, 512, 512
_GD, _GH, _GW = 8, 16, 16
_C12 = 12

_H_SC = 192
_H_TC = _H - _H_SC

_ROWS = 16


def _strip_j0(s):
    return jnp.clip((s - 1) // 2, 0, _GH - 2)


def _tc_kernel(g0_ref, g1_ref, guide_ref, img_ref, out_ref):
    s = pl.program_id(1)
    f32 = jnp.float32

    i_idx = jax.lax.broadcasted_iota(jnp.int32, (_GW, _W), 0).astype(f32)
    x_idx = jax.lax.broadcasted_iota(jnp.int32, (_GW, _W), 1).astype(f32)
    gx = jnp.clip((x_idx + 0.5) * (_GW / _W), 0.5, _GW - 0.5)
    xwt = jnp.maximum(1.0 - jnp.abs(i_idx + 0.5 - gx), 0.0)
    dn = (((1,), (0,)), ((), ()))
    u0 = jax.lax.dot_general(g0_ref[0, 0], xwt, dn,
                             precision=jax.lax.Precision.HIGHEST,
                             preferred_element_type=f32)
    u1 = jax.lax.dot_general(g1_ref[0, 0], xwt, dn,
                             precision=jax.lax.Precision.HIGHEST,
                             preferred_element_type=f32)

    r = jax.lax.broadcasted_iota(jnp.int32, (_ROWS, 1), 0).astype(f32)
    y = s.astype(f32) * _ROWS + r
    gy = jnp.clip((y + 0.5) * (_GH / _H), 0.5, _GH - 0.5)
    fy = jnp.clip(jnp.floor(gy - 0.5), 0.0, _GH - 2.0)
    wy1 = gy - 0.5 - fy
    wy0 = 1.0 - wy1

    gz = jnp.clip(guide_ref[0] * _GD, 0.5, _GD - 0.5)

    a0s, a1s = [], []
    for z in range(_GD):
        zw = jnp.maximum(1.0 - jnp.abs(z + 0.5 - gz), 0.0)
        a0 = zw * wy0
        a0s.append(a0)
        a1s.append(zw - a0)

    img = img_ref[0]
    for o in range(3):
        res = None
        for ci in range(4):
            c = 4 * o + ci
            acc = None
            for z in range(_GD):
                row = z * _C12 + c
                t = a0s[z] * u0[row] + a1s[z] * u1[row]
                acc = t if acc is None else acc + t
            term = acc if ci == 3 else acc * img[ci]
            res = term if res is None else res + term
        out_ref[0, o] = res


def _tc_part(g3, guide, image):
    return pl.pallas_call(
        _tc_kernel,
        grid=(_B, _H_TC // _ROWS),
        in_specs=[
            pl.BlockSpec((1, 1, _GD * _C12, _GW),
                         lambda b, s: (b, _strip_j0(s), 0, 0)),
            pl.BlockSpec((1, 1, _GD * _C12, _GW),
                         lambda b, s: (b, _strip_j0(s) + 1, 0, 0)),
            pl.BlockSpec((1, _ROWS, _W), lambda b, s: (b, s, 0)),
            pl.BlockSpec((1, 3, _ROWS, _W), lambda b, s: (b, 0, s, 0)),
        ],
        out_specs=pl.BlockSpec((1, 3, _ROWS, _W), lambda b, s: (b, 0, s, 0)),
        out_shape=jax.ShapeDtypeStruct((_B, 3, _H, _W), jnp.float32),
        compiler_params=pltpu.CompilerParams(
            dimension_semantics=("parallel", "parallel")),
    )(g3, g3, guide, image)


_ZIC = _GD * _GW * _C12
_NW = 32
_RPW = (_B * _H_SC) // _NW
_CHUNK = 8
_CW = _CHUNK * _W
_NV = _W // 16


def _sc_body(grid_hbm, guide_hbm, image_hbm, out_hbm,
             g_v, gy_v, gd_v, im_v, ob_v):
    f32, i32 = jnp.float32, jnp.int32
    wid = lax.axis_index("s") * 2 + lax.axis_index("c")
    row0 = wid * _RPW
    b = row0 // _H_SC

    pltpu.sync_copy(grid_hbm.at[pl.ds(b * _GH * _ZIC, _GH * _ZIC)], g_v)

    def chunk_body(k, _):
        ysc = row0 % _H_SC + k * _CHUNK
        y0 = _H_TC + ysc
        base_px = (b * _H_SC + ysc) * _W
        pltpu.sync_copy(guide_hbm.at[pl.ds(base_px, _CW)], gd_v)
        for ci in range(3):
            pltpu.sync_copy(
                image_hbm.at[pl.ds(((b * 3 + ci) * _H_SC + ysc) * _W, _CW)],
                im_v.at[pl.ds(ci * _CW, _CW)])

        def row_body(r, _):
            y = y0 + r
            j0 = jnp.clip((y // 16 - 1) // 2, 0, _GH - 2)
            gy = jnp.clip((y.astype(f32) + 0.5) * (_GH / _H), 0.5, _GH - 0.5)
            wy1 = gy - 0.5 - j0.astype(f32)
            wy1v = jnp.full((16,), wy1, f32)
            wy0v = 1.0 - wy1v
            goff = j0 * _ZIC

            def prep_body(k2, _):
                o = k2 * 16
                gy_v[pl.ds(o, 16)] = (
                    wy0v * g_v[pl.ds(goff + o, 16)]
                    + wy1v * g_v[pl.ds(goff + _ZIC + o, 16)])
                return ()
            lax.fori_loop(0, _ZIC // 16, prep_body, (), unroll=8)

            lane = lax.iota(i32, 16)
            roff = r * _W

            def vec_body(v, _):
                px = pl.ds(roff + v * 16, 16)
                i0 = jnp.clip((v - 1) // 2, 0, _GW - 2)
                x = (v * 16 + lane).astype(f32)
                gx = jnp.clip((x + 0.5) * (_GW / _W), 0.5, _GW - 0.5)
                wx1 = gx - 0.5 - i0.astype(f32)
                wx0 = 1.0 - wx1

                g = gd_v[px]
                gz = jnp.clip(g * _GD, 0.5, _GD - 0.5)
                u = gz - 0.5
                fz = jnp.minimum(u.astype(i32), _GD - 2)
                t = u - fz.astype(f32)
                s = 1.0 - t

                base = fz + i0 * (_C12 * _GD)
                co = []
                for c in range(12):
                    g00 = plsc.load_gather(gy_v, [base + c * _GD])
                    g01 = plsc.load_gather(gy_v, [base + (c * _GD + 1)])
                    g10 = plsc.load_gather(gy_v, [base + (c * _GD + _C12 * _GD)])
                    g11 = plsc.load_gather(gy_v, [base + (c * _GD + _C12 * _GD + 1)])
                    v0 = s * g00 + t * g01
                    v1 = s * g10 + t * g11
                    co.append(wx0 * v0 + wx1 * v1)

                p0 = im_v[pl.ds(0 * _CW + roff + v * 16, 16)]
                p1 = im_v[pl.ds(1 * _CW + roff + v * 16, 16)]
                p2 = im_v[pl.ds(2 * _CW + roff + v * 16, 16)]
                for o in range(3):
                    res = (co[4 * o] * p0 + co[4 * o + 1] * p1
                           + co[4 * o + 2] * p2 + co[4 * o + 3])
                    ob_v[pl.ds(o * _CW + roff + v * 16, 16)] = res
                return ()
            lax.fori_loop(0, _NV, vec_body, (), unroll=4)
            return ()
        lax.fori_loop(0, _CHUNK, row_body, ())

        for o in range(3):
            pltpu.sync_copy(
                ob_v.at[pl.ds(o * _CW, _CW)],
                out_hbm.at[pl.ds(((b * 3 + o) * _H_SC + ysc) * _W, _CW)])
        return ()
    lax.fori_loop(0, _RPW // _CHUNK, chunk_body, ())


def _sc_part(g_sc, guide_f, image_f):
    mesh = plsc.VectorSubcoreMesh(core_axis_name="c", subcore_axis_name="s")
    run = pl.kernel(
        _sc_body,
        out_type=jax.ShapeDtypeStruct((_B * 3 * _H_SC * _W,), jnp.float32),
        mesh=mesh,
        compiler_params=pltpu.CompilerParams(needs_layout_passes=False),
        scratch_types=[
            pltpu.VMEM((_GH * _ZIC,), jnp.float32),
            pltpu.VMEM((_ZIC,), jnp.float32),
            pltpu.VMEM((_CW,), jnp.float32),
            pltpu.VMEM((3 * _CW,), jnp.float32),
            pltpu.VMEM((3 * _CW,), jnp.float32),
        ],
    )
    return run(g_sc, guide_f, image_f)


@jax.jit
def kernel(grid, guide, image):
    B, C12, gd, gh, gw = grid.shape
    g3 = jnp.transpose(grid, (0, 3, 2, 1, 4)).reshape(B, gh, gd * C12, gw)
    g_sc = jnp.transpose(grid, (0, 3, 4, 1, 2)).reshape(B * gh * gd * gw * C12)
    guide_f = guide[:, _H_TC:, :].reshape(B * _H_SC * _W)
    image_f = image[:, :, _H_TC:, :].reshape(B * 3 * _H_SC * _W)

    tc_out = _tc_part(g3, guide, image)
    sc_out = _sc_part(g_sc, guide_f, image_f).reshape(B, 3, _H_SC, _W)
    return lax.dynamic_update_slice(tc_out, sc_out, (0, 0, _H_TC, 0))


# speedup vs baseline: 1.7570x; 1.0349x over previous
import jax
import jax.numpy as jnp
from jax import lax
from jax.experimental import pallas as pl
from jax.experimental.pallas import tpu as pltpu
from jax.experimental.pallas import tpu_sc as plsc

_B, _H, _W = 4, 512, 512
_GD, _GH, _GW = 8, 16, 16
_C12 = 12

_H_SC = 192
_H_TC = _H - _H_SC

_ROWS = 16


def _strip_j0(s):
    return jnp.clip((s - 1) // 2, 0, _GH - 2)


def _tc_kernel(g0_ref, g1_ref, guide_ref, img_ref, out_ref):
    s = pl.program_id(1)
    f32 = jnp.float32

    i_idx = jax.lax.broadcasted_iota(jnp.int32, (_GW, _W), 0).astype(f32)
    x_idx = jax.lax.broadcasted_iota(jnp.int32, (_GW, _W), 1).astype(f32)
    gx = jnp.clip((x_idx + 0.5) * (_GW / _W), 0.5, _GW - 0.5)
    xwt = jnp.maximum(1.0 - jnp.abs(i_idx + 0.5 - gx), 0.0)
    dn = (((1,), (0,)), ((), ()))
    u0 = jax.lax.dot_general(g0_ref[0, 0], xwt, dn,
                             precision=jax.lax.Precision.HIGHEST,
                             preferred_element_type=f32)
    u1 = jax.lax.dot_general(g1_ref[0, 0], xwt, dn,
                             precision=jax.lax.Precision.HIGHEST,
                             preferred_element_type=f32)

    r = jax.lax.broadcasted_iota(jnp.int32, (_ROWS, 1), 0).astype(f32)
    y = s.astype(f32) * _ROWS + r
    gy = jnp.clip((y + 0.5) * (_GH / _H), 0.5, _GH - 0.5)
    fy = jnp.clip(jnp.floor(gy - 0.5), 0.0, _GH - 2.0)
    wy1 = gy - 0.5 - fy
    wy0 = 1.0 - wy1

    gz = jnp.clip(guide_ref[0] * _GD, 0.5, _GD - 0.5)

    a0s, a1s = [], []
    for z in range(_GD):
        zw = jnp.maximum(1.0 - jnp.abs(z + 0.5 - gz), 0.0)
        a0 = zw * wy0
        a0s.append(a0)
        a1s.append(zw - a0)

    img = img_ref[0]
    for o in range(3):
        res = None
        for ci in range(4):
            c = 4 * o + ci
            acc = None
            for z in range(_GD):
                row = z * _C12 + c
                t = a0s[z] * u0[row] + a1s[z] * u1[row]
                acc = t if acc is None else acc + t
            term = acc if ci == 3 else acc * img[ci]
            res = term if res is None else res + term
        out_ref[0, o] = res


def _tc_part(g3, guide, image):
    return pl.pallas_call(
        _tc_kernel,
        grid=(_B, _H_TC // _ROWS),
        in_specs=[
            pl.BlockSpec((1, 1, _GD * _C12, _GW),
                         lambda b, s: (b, _strip_j0(s), 0, 0)),
            pl.BlockSpec((1, 1, _GD * _C12, _GW),
                         lambda b, s: (b, _strip_j0(s) + 1, 0, 0)),
            pl.BlockSpec((1, _ROWS, _W), lambda b, s: (b, s, 0)),
            pl.BlockSpec((1, 3, _ROWS, _W), lambda b, s: (b, 0, s, 0)),
        ],
        out_specs=pl.BlockSpec((1, 3, _ROWS, _W), lambda b, s: (b, 0, s, 0)),
        out_shape=jax.ShapeDtypeStruct((_B, 3, _H, _W), jnp.float32),
        compiler_params=pltpu.CompilerParams(
            dimension_semantics=("parallel", "parallel")),
    )(g3, g3, guide, image)


_ZIC = _GD * _GW * _C12
_NW = 32
_RPW = (_B * _H_SC) // _NW
_CHUNK = 8
_CW = _CHUNK * _W
_NV = _W // 16


def _sc_body(grid_hbm, guide_hbm, image_hbm, out_hbm,
             g_v, gy_v, gd_v, im_v, ob_v):
    f32, i32 = jnp.float32, jnp.int32
    wid = lax.axis_index("s") * 2 + lax.axis_index("c")
    row0 = wid * _RPW
    b = row0 // _H_SC

    pltpu.sync_copy(grid_hbm.at[pl.ds(b * _GH * _ZIC, _GH * _ZIC)], g_v)

    def chunk_body(k, _):
        ysc = row0 % _H_SC + k * _CHUNK
        y0 = _H_TC + ysc
        pltpu.sync_copy(guide_hbm.at[pl.ds(b * _H_SC + ysc, _CHUNK), :], gd_v)
        for ci in range(3):
            pltpu.sync_copy(
                image_hbm.at[pl.ds((b * 3 + ci) * _H_SC + ysc, _CHUNK), :],
                im_v.at[ci])

        def row_body(r, _):
            y = y0 + r
            j0 = jnp.clip((y // 16 - 1) // 2, 0, _GH - 2)
            gy = jnp.clip((y.astype(f32) + 0.5) * (_GH / _H), 0.5, _GH - 0.5)
            wy1 = gy - 0.5 - j0.astype(f32)
            wy1v = jnp.full((16,), wy1, f32)
            wy0v = 1.0 - wy1v
            goff = j0 * _ZIC

            def prep_body(k2, _):
                o = k2 * 16
                gy_v[pl.ds(o, 16)] = (
                    wy0v * g_v[pl.ds(goff + o, 16)]
                    + wy1v * g_v[pl.ds(goff + _ZIC + o, 16)])
                return ()
            lax.fori_loop(0, _ZIC // 16, prep_body, (), unroll=8)

            lane = lax.iota(i32, 16)
            roff = r * _W

            def vec_body(v, _):
                px = pl.ds(roff + v * 16, 16)
                i0 = jnp.clip((v - 1) // 2, 0, _GW - 2)
                x = (v * 16 + lane).astype(f32)
                gx = jnp.clip((x + 0.5) * (_GW / _W), 0.5, _GW - 0.5)
                wx1 = gx - 0.5 - i0.astype(f32)
                wx0 = 1.0 - wx1

                tr = r
                tcl = v * 16
                g = gd_v[tr, pl.ds(tcl, 16)]
                gz = jnp.clip(g * _GD, 0.5, _GD - 0.5)
                u = gz - 0.5
                fz = jnp.minimum(u.astype(i32), _GD - 2)
                t = u - fz.astype(f32)
                s = 1.0 - t

                base = fz + i0 * (_C12 * _GD)
                co = []
                for c in range(12):
                    g00 = plsc.load_gather(gy_v, [base + c * _GD])
                    g01 = plsc.load_gather(gy_v, [base + (c * _GD + 1)])
                    g10 = plsc.load_gather(gy_v, [base + (c * _GD + _C12 * _GD)])
                    g11 = plsc.load_gather(gy_v, [base + (c * _GD + _C12 * _GD + 1)])
                    v0 = s * g00 + t * g01
                    v1 = s * g10 + t * g11
                    co.append(wx0 * v0 + wx1 * v1)

                p0 = im_v[0, tr, pl.ds(tcl, 16)]
                p1 = im_v[1, tr, pl.ds(tcl, 16)]
                p2 = im_v[2, tr, pl.ds(tcl, 16)]
                for o in range(3):
                    res = (co[4 * o] * p0 + co[4 * o + 1] * p1
                           + co[4 * o + 2] * p2 + co[4 * o + 3])
                    ob_v[o, tr, pl.ds(tcl, 16)] = res
                return ()
            lax.fori_loop(0, _NV, vec_body, (), unroll=4)
            return ()
        lax.fori_loop(0, _CHUNK, row_body, ())

        for o in range(3):
            pltpu.sync_copy(
                ob_v.at[o],
                out_hbm.at[pl.ds((b * 3 + o) * _H_SC + ysc, _CHUNK), :])
        return ()
    lax.fori_loop(0, _RPW // _CHUNK, chunk_body, ())


def _sc_part(g_sc, guide_f, image_f):
    mesh = plsc.VectorSubcoreMesh(core_axis_name="c", subcore_axis_name="s")
    run = pl.kernel(
        _sc_body,
        out_type=jax.ShapeDtypeStruct((_B * 3 * _H_SC, _W), jnp.float32),
        mesh=mesh,
        compiler_params=pltpu.CompilerParams(needs_layout_passes=False),
        scratch_types=[
            pltpu.VMEM((_GH * _ZIC,), jnp.float32),
            pltpu.VMEM((_ZIC,), jnp.float32),
            pltpu.VMEM((_CHUNK, _W), jnp.float32),
            pltpu.VMEM((3, _CHUNK, _W), jnp.float32),
            pltpu.VMEM((3, _CHUNK, _W), jnp.float32),
        ],
    )
    return run(g_sc, guide_f, image_f)


@jax.jit
def kernel(grid, guide, image):
    B, C12, gd, gh, gw = grid.shape
    g3 = jnp.transpose(grid, (0, 3, 2, 1, 4)).reshape(B, gh, gd * C12, gw)
    g_sc = jnp.transpose(grid, (0, 3, 4, 1, 2)).reshape(B * gh * gd * gw * C12)
    guide_f = guide[:, _H_TC:, :].reshape(B * _H_SC, _W)
    image_f = image[:, :, _H_TC:, :].reshape(B * 3 * _H_SC, _W)

    tc_out = _tc_part(g3, guide, image)
    sc_out = _sc_part(g_sc, guide_f, image_f).reshape(B, 3, _H_SC, _W)
    return lax.dynamic_update_slice(tc_out, sc_out, (0, 0, _H_TC, 0))


# speedup vs baseline: 1.8480x; 1.0518x over previous
import jax
import jax.numpy as jnp
from jax import lax
from jax.experimental import pallas as pl
from jax.experimental.pallas import tpu as pltpu
from jax.experimental.pallas import tpu_sc as plsc

_B, _H, _W = 4, 512, 512
_GD, _GH, _GW = 8, 16, 16
_C12 = 12

_H_SC = 192
_H_TC = _H - _H_SC

_ROWS = 16


def _strip_j0(s):
    return jnp.clip((s - 1) // 2, 0, _GH - 2)


def _tc_kernel(g0_ref, g1_ref, guide_ref, img_ref, out_ref):
    s = pl.program_id(1)
    f32 = jnp.float32

    i_idx = jax.lax.broadcasted_iota(jnp.int32, (_GW, _W), 0).astype(f32)
    x_idx = jax.lax.broadcasted_iota(jnp.int32, (_GW, _W), 1).astype(f32)
    gx = jnp.clip((x_idx + 0.5) * (_GW / _W), 0.5, _GW - 0.5)
    xwt = jnp.maximum(1.0 - jnp.abs(i_idx + 0.5 - gx), 0.0)
    dn = (((1,), (0,)), ((), ()))
    u0 = jax.lax.dot_general(g0_ref[0, 0], xwt, dn,
                             precision=jax.lax.Precision.HIGHEST,
                             preferred_element_type=f32)
    u1 = jax.lax.dot_general(g1_ref[0, 0], xwt, dn,
                             precision=jax.lax.Precision.HIGHEST,
                             preferred_element_type=f32)

    r = jax.lax.broadcasted_iota(jnp.int32, (_ROWS, 1), 0).astype(f32)
    y = s.astype(f32) * _ROWS + r
    gy = jnp.clip((y + 0.5) * (_GH / _H), 0.5, _GH - 0.5)
    fy = jnp.clip(jnp.floor(gy - 0.5), 0.0, _GH - 2.0)
    wy1 = gy - 0.5 - fy
    wy0 = 1.0 - wy1

    gz = jnp.clip(guide_ref[0] * _GD, 0.5, _GD - 0.5)

    a0s, a1s = [], []
    for z in range(_GD):
        zw = jnp.maximum(1.0 - jnp.abs(z + 0.5 - gz), 0.0)
        a0 = zw * wy0
        a0s.append(a0)
        a1s.append(zw - a0)

    img = img_ref[0]
    for o in range(3):
        res = None
        for ci in range(4):
            c = 4 * o + ci
            acc = None
            for z in range(_GD):
                row = z * _C12 + c
                t = a0s[z] * u0[row] + a1s[z] * u1[row]
                acc = t if acc is None else acc + t
            term = acc if ci == 3 else acc * img[ci]
            res = term if res is None else res + term
        out_ref[0, o] = res


def _tc_part(g3, guide, image):
    return pl.pallas_call(
        _tc_kernel,
        grid=(_B, _H_TC // _ROWS),
        in_specs=[
            pl.BlockSpec((1, 1, _GD * _C12, _GW),
                         lambda b, s: (b, _strip_j0(s), 0, 0)),
            pl.BlockSpec((1, 1, _GD * _C12, _GW),
                         lambda b, s: (b, _strip_j0(s) + 1, 0, 0)),
            pl.BlockSpec((1, _ROWS, _W), lambda b, s: (b, s, 0)),
            pl.BlockSpec((1, 3, _ROWS, _W), lambda b, s: (b, 0, s, 0)),
        ],
        out_specs=pl.BlockSpec((1, 3, _ROWS, _W), lambda b, s: (b, 0, s, 0)),
        out_shape=jax.ShapeDtypeStruct((_B, 3, _H, _W), jnp.float32),
        compiler_params=pltpu.CompilerParams(
            dimension_semantics=("parallel", "parallel")),
    )(g3, g3, guide, image)


_ZIC = _GD * _GW * _C12
_NW = 32
_RPW = (_B * _H_SC) // _NW
_CHUNK = 8
_CW = _CHUNK * _W
_NV = _W // 16


def _sc_body(grid_hbm, guide_hbm, image_hbm, out_hbm,
             g_v, gy_v, gd_v, im_v, ob_v):
    f32, i32 = jnp.float32, jnp.int32
    wid = lax.axis_index("s") * 2 + lax.axis_index("c")
    row0 = wid * _RPW
    b = row0 // _H_SC

    pltpu.sync_copy(grid_hbm.at[pl.ds(b * _GH * _ZIC, _GH * _ZIC)], g_v)

    def chunk_body(k, _):
        ysc = row0 % _H_SC + k * _CHUNK
        y0 = _H_TC + ysc
        pltpu.sync_copy(guide_hbm.at[pl.ds(b * _H + y0, _CHUNK), :], gd_v)
        for ci in range(3):
            pltpu.sync_copy(
                image_hbm.at[pl.ds((b * 3 + ci) * _H + y0, _CHUNK), :],
                im_v.at[ci])

        def row_body(r, _):
            y = y0 + r
            j0 = jnp.clip((y // 16 - 1) // 2, 0, _GH - 2)
            gy = jnp.clip((y.astype(f32) + 0.5) * (_GH / _H), 0.5, _GH - 0.5)
            wy1 = gy - 0.5 - j0.astype(f32)
            wy1v = jnp.full((16,), wy1, f32)
            wy0v = 1.0 - wy1v
            goff = j0 * _ZIC

            def prep_body(k2, _):
                o = k2 * 16
                gy_v[pl.ds(o, 16)] = (
                    wy0v * g_v[pl.ds(goff + o, 16)]
                    + wy1v * g_v[pl.ds(goff + _ZIC + o, 16)])
                return ()
            lax.fori_loop(0, _ZIC // 16, prep_body, (), unroll=8)

            lane = lax.iota(i32, 16)
            roff = r * _W

            def vec_body(v, _):
                px = pl.ds(roff + v * 16, 16)
                i0 = jnp.clip((v - 1) // 2, 0, _GW - 2)
                x = (v * 16 + lane).astype(f32)
                gx = jnp.clip((x + 0.5) * (_GW / _W), 0.5, _GW - 0.5)
                wx1 = gx - 0.5 - i0.astype(f32)
                wx0 = 1.0 - wx1

                tr = r
                tcl = v * 16
                g = gd_v[tr, pl.ds(tcl, 16)]
                gz = jnp.clip(g * _GD, 0.5, _GD - 0.5)
                u = gz - 0.5
                fz = jnp.minimum(u.astype(i32), _GD - 2)
                t = u - fz.astype(f32)
                s = 1.0 - t

                base = fz + i0 * (_C12 * _GD)
                co = []
                for c in range(12):
                    g00 = plsc.load_gather(gy_v, [base + c * _GD])
                    g01 = plsc.load_gather(gy_v, [base + (c * _GD + 1)])
                    g10 = plsc.load_gather(gy_v, [base + (c * _GD + _C12 * _GD)])
                    g11 = plsc.load_gather(gy_v, [base + (c * _GD + _C12 * _GD + 1)])
                    v0 = s * g00 + t * g01
                    v1 = s * g10 + t * g11
                    co.append(wx0 * v0 + wx1 * v1)

                p0 = im_v[0, tr, pl.ds(tcl, 16)]
                p1 = im_v[1, tr, pl.ds(tcl, 16)]
                p2 = im_v[2, tr, pl.ds(tcl, 16)]
                for o in range(3):
                    res = (co[4 * o] * p0 + co[4 * o + 1] * p1
                           + co[4 * o + 2] * p2 + co[4 * o + 3])
                    ob_v[o, tr, pl.ds(tcl, 16)] = res
                return ()
            lax.fori_loop(0, _NV, vec_body, (), unroll=4)
            return ()
        lax.fori_loop(0, _CHUNK, row_body, ())

        for o in range(3):
            pltpu.sync_copy(
                ob_v.at[o],
                out_hbm.at[pl.ds((b * 3 + o) * _H_SC + ysc, _CHUNK), :])
        return ()
    lax.fori_loop(0, _RPW // _CHUNK, chunk_body, ())


def _sc_part(g_sc, guide_f, image_f):
    mesh = plsc.VectorSubcoreMesh(core_axis_name="c", subcore_axis_name="s")
    run = pl.kernel(
        _sc_body,
        out_type=jax.ShapeDtypeStruct((_B * 3 * _H_SC, _W), jnp.float32),
        mesh=mesh,
        compiler_params=pltpu.CompilerParams(needs_layout_passes=False),
        scratch_types=[
            pltpu.VMEM((_GH * _ZIC,), jnp.float32),
            pltpu.VMEM((_ZIC,), jnp.float32),
            pltpu.VMEM((_CHUNK, _W), jnp.float32),
            pltpu.VMEM((3, _CHUNK, _W), jnp.float32),
            pltpu.VMEM((3, _CHUNK, _W), jnp.float32),
        ],
    )
    return run(g_sc, guide_f, image_f)


@jax.jit
def kernel(grid, guide, image):
    B, C12, gd, gh, gw = grid.shape
    g3 = jnp.transpose(grid, (0, 3, 2, 1, 4)).reshape(B, gh, gd * C12, gw)
    g_sc = jnp.transpose(grid, (0, 3, 4, 1, 2)).reshape(B * gh * gd * gw * C12)
    guide_f = guide.reshape(B * _H, _W)
    image_f = image.reshape(B * 3 * _H, _W)

    tc_out = _tc_part(g3, guide, image)
    sc_out = _sc_part(g_sc, guide_f, image_f).reshape(B, 3, _H_SC, _W)
    return lax.dynamic_update_slice(tc_out, sc_out, (0, 0, _H_TC, 0))
